# baseline (speedup 1.0000x reference)
import jax
import jax.numpy as jnp
from jax import lax
from jax.experimental import pallas as pl
from jax.experimental.pallas import tpu as pltpu
from jax.experimental.pallas import tpu_sc as plsc

N_NODES = 10000
N_PAD = 10112
N_EDGES = 320000
E_PAD = 327680
D = 128

NC, NS = 2, 16
NW = NC * NS
EB = 128
EROWS = E_PAD // NW // EB
DEG_ROWS = E_PAD // NS // EB
RPT = N_PAD // NS
DEG_Q = 8

_MESH = plsc.VectorSubcoreMesh(core_axis_name="c", subcore_axis_name="s")


def _deg_body(ei, degs, idx_v, ones_v, zeros_v, sd, acc):
    c = lax.axis_index("c")
    s = lax.axis_index("s")
    zv = jnp.zeros((16,), jnp.float32)
    ov = jnp.ones((16,), jnp.float32)
    for j in range(64 // 16):
        zeros_v[pl.ds(j * 16, 16)] = zv
    for j in range(EB // 16):
        ones_v[pl.ds(j * 16, 16)] = ov
    for m in range(9):
        pltpu.sync_copy(zeros_v, acc.at[pl.ds(s * RPT + m * 64, 64)])
    pltpu.sync_copy(zeros_v.at[pl.ds(0, 56)], acc.at[pl.ds(s * RPT + 576, 56)])
    plsc.subcore_barrier()
    pltpu.sync_copy(ei.at[pl.ds(s * DEG_ROWS, DEG_ROWS)], idx_v)

    def body(j, carry):
        for b in range(DEG_Q):
            k = j * DEG_Q + b
            pltpu.async_copy(ones_v, acc.at[idx_v.at[k, c]], sd, add=True)
        for b in range(DEG_Q):
            k = j * DEG_Q + b
            pltpu.make_async_copy(ones_v, acc.at[idx_v.at[k, c]], sd).wait()
        return carry

    lax.fori_loop(0, DEG_ROWS // DEG_Q, body, 0)
    plsc.subcore_barrier()
    pltpu.sync_copy(acc.at[pl.ds(s * RPT, RPT)], degs.at[c, pl.ds(s * RPT, RPT)])


_deg_kernel = pl.kernel(
    _deg_body,
    out_type=jax.ShapeDtypeStruct((NC, N_PAD), jnp.float32),
    mesh=_MESH,
    compiler_params=pltpu.CompilerParams(use_tc_tiling_on_sc=False),
    scratch_types=[
        pltpu.VMEM((DEG_ROWS, 2, EB), jnp.int32),
        pltpu.VMEM((EB,), jnp.float32),
        pltpu.VMEM((64,), jnp.float32),
        pltpu.SemaphoreType.DMA,
        pltpu.VMEM_SHARED((N_PAD,), jnp.float32),
    ],
)


def _agg_body(h, ei, out, idxr, rows,
              si0, si1, si2, sg0, sg1, sg2, ss0, ss1, ss2, acc):
    c = lax.axis_index("c")
    s = lax.axis_index("s")
    w = c * NS + s
    base = w * EROWS
    si = [si0, si1, si2]
    sg = [sg0, sg1, sg2]
    ss = [ss0, ss1, ss2]
    zv = jnp.zeros((16,), jnp.float32)

    def zero_rows(i, carry):
        for j in range(D // 16):
            rows[0, i, pl.ds(j * 16, 16)] = zv
        return carry

    lax.fori_loop(0, EB, zero_rows, 0)
    for m in range(4):
        pltpu.sync_copy(rows.at[0], acc.at[pl.ds(s * RPT + m * EB, EB)])
    pltpu.sync_copy(rows.at[0, pl.ds(0, 120)],
                    acc.at[pl.ds(s * RPT + 512, 120)])
    plsc.subcore_barrier()

    def idx_load(k, b):
        pltpu.async_copy(ei.at[base + k], idxr.at[b], si[b])

    def idx_wait(k, b):
        pltpu.make_async_copy(ei.at[base + k], idxr.at[b], si[b]).wait()

    def g_issue(b):
        pltpu.async_copy(h.at[idxr.at[b, 0]], rows.at[b], sg[b])

    def g_wait(b):
        pltpu.make_async_copy(h.at[idxr.at[b, 0]], rows.at[b], sg[b]).wait()

    def s_issue(b):
        pltpu.async_copy(rows.at[b], acc.at[idxr.at[b, 1]], ss[b], add=True)

    def s_wait(b):
        pltpu.make_async_copy(rows.at[b], acc.at[idxr.at[b, 1]], ss[b]).wait()

    for b in range(3):
        idx_load(b, b)
    for b in range(3):
        idx_wait(b, b)
        g_issue(b)
        g_wait(b)
        s_issue(b)
        idx_load(b + 3, b)

    def body(j, carry):
        for b in range(3):
            k = 3 * j + b
            s_wait(b)
            idx_wait(k, b)
            g_issue(b)
            g_wait(b)
            s_issue(b)
            idx_load(k + 3, b)
        return carry

    lax.fori_loop(1, EROWS // 3 - 1, body, 0)
    for k in range(EROWS - 5, EROWS):
        b = k % 3
        s_wait(b)
        idx_wait(k, b)
        g_issue(b)
        g_wait(b)
        s_issue(b)
        if k + 3 < EROWS:
            idx_load(k + 3, b)
    for b in (2, 0, 1):
        s_wait(b)
    plsc.subcore_barrier()
    pltpu.sync_copy(acc.at[pl.ds(s * RPT, RPT)], out.at[c, pl.ds(s * RPT, RPT)])


_agg_kernel = pl.kernel(
    _agg_body,
    out_type=jax.ShapeDtypeStruct((NC, N_PAD, D), jnp.float32),
    mesh=_MESH,
    compiler_params=pltpu.CompilerParams(use_tc_tiling_on_sc=False),
    scratch_types=[
        pltpu.VMEM((3, 2, EB), jnp.int32),
        pltpu.VMEM((3, EB, D), jnp.float32),
        pltpu.SemaphoreType.DMA,
        pltpu.SemaphoreType.DMA,
        pltpu.SemaphoreType.DMA,
        pltpu.SemaphoreType.DMA,
        pltpu.SemaphoreType.DMA,
        pltpu.SemaphoreType.DMA,
        pltpu.SemaphoreType.DMA,
        pltpu.SemaphoreType.DMA,
        pltpu.SemaphoreType.DMA,
        pltpu.VMEM_SHARED((N_PAD, D), jnp.float32),
    ],
)


def _norm(deg):
    return jnp.where(deg > 0, lax.rsqrt(jnp.maximum(deg, 1.0)), 0.0)


def _mm1_body(x_ref, dgo_ref, w_ref, o_ref):
    nsrc = _norm(dgo_ref[...])
    o_ref[...] = jnp.dot(x_ref[...] * nsrc, w_ref[...],
                         preferred_element_type=jnp.float32,
                         precision=lax.Precision.HIGHEST)


def _mid_body(a0_ref, a1_ref, dgo_ref, dgi_ref, b1_ref, w_ref, o_ref):
    ndst = _norm(dgi_ref[...])
    nsrc = _norm(dgo_ref[...])
    t = jnp.maximum((a0_ref[...] + a1_ref[...]) * ndst + b1_ref[...], 0.0)
    o_ref[...] = jnp.dot(t * nsrc, w_ref[...],
                         preferred_element_type=jnp.float32,
                         precision=lax.Precision.HIGHEST)


def _fin_body(a0_ref, a1_ref, dgi_ref, b2_ref, o_ref):
    ndst = _norm(dgi_ref[...])
    o_ref[...] = (a0_ref[...] + a1_ref[...]) * ndst + b2_ref[...]


_R = 632
_GRID = (N_PAD // _R,)
_row_spec = pl.BlockSpec((_R, D), lambda i: (i, 0))
_deg_spec = pl.BlockSpec((_R, 1), lambda i: (i, 0))
_w_spec = pl.BlockSpec((D, D), lambda i: (0, 0))
_b_spec = pl.BlockSpec((1, D), lambda i: (0, 0))
_out_struct = jax.ShapeDtypeStruct((N_PAD, D), jnp.float32)

_mm1 = pl.pallas_call(
    _mm1_body, grid=_GRID,
    in_specs=[_row_spec, _deg_spec, _w_spec],
    out_specs=_row_spec, out_shape=_out_struct)

_mid = pl.pallas_call(
    _mid_body, grid=_GRID,
    in_specs=[_row_spec, _row_spec, _deg_spec, _deg_spec, _b_spec, _w_spec],
    out_specs=_row_spec, out_shape=_out_struct)

_FR = 1000
_fin = pl.pallas_call(
    _fin_body, grid=(N_NODES // _FR,),
    in_specs=[pl.BlockSpec((_FR, D), lambda i: (i, 0)),
              pl.BlockSpec((_FR, D), lambda i: (i, 0)),
              pl.BlockSpec((_FR, 1), lambda i: (i, 0)),
              pl.BlockSpec((1, D), lambda i: (0, 0))],
    out_specs=pl.BlockSpec((_FR, D), lambda i: (i, 0)),
    out_shape=jax.ShapeDtypeStruct((N_NODES, D), jnp.float32))


def kernel(features, edge_index, W1, b1, W2, b2):
    ei32 = edge_index.astype(jnp.int32)
    pad = N_NODES + jnp.arange(E_PAD - N_EDGES, dtype=jnp.int32) % (N_PAD - N_NODES)
    ei = jnp.concatenate([ei32, jnp.stack([pad, pad])], axis=1)
    ei = ei.reshape(2, E_PAD // EB, EB).transpose(1, 0, 2)
    x = jnp.pad(features.astype(jnp.float32), ((0, N_PAD - N_NODES), (0, 0)))
    degs = _deg_kernel(ei)
    dgo = degs[0].reshape(N_PAD, 1)
    dgi = degs[1].reshape(N_PAD, 1)
    h1s = _mm1(x, dgo, W1)
    agg1 = _agg_kernel(h1s, ei)
    h2s = _mid(agg1[0], agg1[1], dgo, dgi, b1.reshape(1, D), W2)
    agg2 = _agg_kernel(h2s, ei)
    return _fin(agg2[0], agg2[1], dgi, b2.reshape(1, D))


# speedup vs baseline: 1.0530x; 1.0530x over previous
import jax
import jax.numpy as jnp
from jax import lax
from jax.experimental import pallas as pl
from jax.experimental.pallas import tpu as pltpu
from jax.experimental.pallas import tpu_sc as plsc

N_NODES = 10000
N_PAD = 10112
N_EDGES = 320000
E_PAD = 327680
D = 128

NC, NS = 2, 16
NW = NC * NS
EB = 128
EROWS = E_PAD // NW // EB
DEG_ROWS = E_PAD // NS // EB
RPT = N_PAD // NS
DEG_Q = 8

_MESH = plsc.VectorSubcoreMesh(core_axis_name="c", subcore_axis_name="s")


def _deg_body(ei, degs, idx_v, ones_v, zeros_v, sd, acc):
    c = lax.axis_index("c")
    s = lax.axis_index("s")
    zv = jnp.zeros((16,), jnp.float32)
    ov = jnp.ones((16,), jnp.float32)
    for j in range(64 // 16):
        zeros_v[pl.ds(j * 16, 16)] = zv
    for j in range(EB // 16):
        ones_v[pl.ds(j * 16, 16)] = ov
    for m in range(9):
        pltpu.sync_copy(zeros_v, acc.at[pl.ds(s * RPT + m * 64, 64)])
    pltpu.sync_copy(zeros_v.at[pl.ds(0, 56)], acc.at[pl.ds(s * RPT + 576, 56)])
    plsc.subcore_barrier()
    pltpu.sync_copy(ei.at[pl.ds(s * DEG_ROWS, DEG_ROWS)], idx_v)

    def body(j, carry):
        for b in range(DEG_Q):
            k = j * DEG_Q + b
            pltpu.async_copy(ones_v, acc.at[idx_v.at[k, c]], sd, add=True)
        for b in range(DEG_Q):
            k = j * DEG_Q + b
            pltpu.make_async_copy(ones_v, acc.at[idx_v.at[k, c]], sd).wait()
        return carry

    lax.fori_loop(0, DEG_ROWS // DEG_Q, body, 0)
    plsc.subcore_barrier()
    pltpu.sync_copy(acc.at[pl.ds(s * RPT, RPT)], degs.at[c, pl.ds(s * RPT, RPT)])


_deg_kernel = pl.kernel(
    _deg_body,
    out_type=jax.ShapeDtypeStruct((NC, N_PAD), jnp.float32),
    mesh=_MESH,
    compiler_params=pltpu.CompilerParams(use_tc_tiling_on_sc=False),
    scratch_types=[
        pltpu.VMEM((DEG_ROWS, 2, EB), jnp.int32),
        pltpu.VMEM((EB,), jnp.float32),
        pltpu.VMEM((64,), jnp.float32),
        pltpu.SemaphoreType.DMA,
        pltpu.VMEM_SHARED((N_PAD,), jnp.float32),
    ],
)


def _agg_body(h, ei, out, idxr, rows,
              si0, si1, si2, sg0, sg1, sg2, ss0, ss1, ss2, acc):
    c = lax.axis_index("c")
    s = lax.axis_index("s")
    w = c * NS + s
    base = w * EROWS
    si = [si0, si1, si2]
    sg = [sg0, sg1, sg2]
    ss = [ss0, ss1, ss2]
    zv = jnp.zeros((16,), jnp.float32)

    def idx_load(k, b):
        pltpu.async_copy(ei.at[base + k], idxr.at[b], si[b])

    def idx_wait(k, b):
        pltpu.make_async_copy(ei.at[base + k], idxr.at[b], si[b]).wait()

    def g_issue(b):
        pltpu.async_copy(h.at[idxr.at[b, 0]], rows.at[b], sg[b])

    def g_wait(b):
        pltpu.make_async_copy(h.at[idxr.at[b, 0]], rows.at[b], sg[b]).wait()

    def s_issue(b):
        pltpu.async_copy(rows.at[b], acc.at[idxr.at[b, 1]], ss[b], add=True)

    def s_wait(b):
        pltpu.make_async_copy(rows.at[b], acc.at[idxr.at[b, 1]], ss[b]).wait()

    for b in range(3):
        idx_load(b, b)

    def zero_rows(i, carry):
        for j in range(D // 16):
            rows[0, i, pl.ds(j * 16, 16)] = zv
        return carry

    lax.fori_loop(0, EB, zero_rows, 0)
    for m in range(4):
        pltpu.sync_copy(rows.at[0], acc.at[pl.ds(s * RPT + m * EB, EB)])
    pltpu.sync_copy(rows.at[0, pl.ds(0, 120)],
                    acc.at[pl.ds(s * RPT + 512, 120)])
    plsc.subcore_barrier()

    for b in range(3):
        idx_wait(b, b)
        g_issue(b)
        g_wait(b)
        s_issue(b)
        idx_load(b + 3, b)

    def body(j, carry):
        for b in range(3):
            k = 3 * j + b
            s_wait(b)
            idx_wait(k, b)
            g_issue(b)
            g_wait(b)
            s_issue(b)
            idx_load(k + 3, b)
        return carry

    lax.fori_loop(1, EROWS // 3 - 1, body, 0)
    for k in range(EROWS - 5, EROWS):
        b = k % 3
        s_wait(b)
        idx_wait(k, b)
        g_issue(b)
        g_wait(b)
        s_issue(b)
        if k + 3 < EROWS:
            idx_load(k + 3, b)
    for b in (2, 0, 1):
        s_wait(b)
    plsc.subcore_barrier()
    pltpu.sync_copy(acc.at[pl.ds(s * RPT, RPT)], out.at[c, pl.ds(s * RPT, RPT)])


_agg_kernel = pl.kernel(
    _agg_body,
    out_type=jax.ShapeDtypeStruct((NC, N_PAD, D), jnp.float32),
    mesh=_MESH,
    compiler_params=pltpu.CompilerParams(use_tc_tiling_on_sc=False),
    scratch_types=[
        pltpu.VMEM((3, 2, EB), jnp.int32),
        pltpu.VMEM((3, EB, D), jnp.float32),
        pltpu.SemaphoreType.DMA,
        pltpu.SemaphoreType.DMA,
        pltpu.SemaphoreType.DMA,
        pltpu.SemaphoreType.DMA,
        pltpu.SemaphoreType.DMA,
        pltpu.SemaphoreType.DMA,
        pltpu.SemaphoreType.DMA,
        pltpu.SemaphoreType.DMA,
        pltpu.SemaphoreType.DMA,
        pltpu.VMEM_SHARED((N_PAD, D), jnp.float32),
    ],
)


def _norm(deg):
    return jnp.where(deg > 0, lax.rsqrt(jnp.maximum(deg, 1.0)), 0.0)


def _mm1_body(x_ref, dgo_ref, w_ref, o_ref):
    nsrc = _norm(dgo_ref[...])
    o_ref[...] = jnp.dot(x_ref[...] * nsrc, w_ref[...],
                         preferred_element_type=jnp.float32)


def _mid_body(a0_ref, a1_ref, dgo_ref, dgi_ref, b1_ref, w_ref, o_ref):
    ndst = _norm(dgi_ref[...])
    nsrc = _norm(dgo_ref[...])
    t = jnp.maximum((a0_ref[...] + a1_ref[...]) * ndst + b1_ref[...], 0.0)
    o_ref[...] = jnp.dot(t * nsrc, w_ref[...],
                         preferred_element_type=jnp.float32)


def _fin_body(a0_ref, a1_ref, dgi_ref, b2_ref, o_ref):
    ndst = _norm(dgi_ref[...])
    o_ref[...] = (a0_ref[...] + a1_ref[...]) * ndst + b2_ref[...]


_R = 1264
_GRID = (N_PAD // _R,)
_row_spec = pl.BlockSpec((_R, D), lambda i: (i, 0))
_deg_spec = pl.BlockSpec((_R, 1), lambda i: (i, 0))
_w_spec = pl.BlockSpec((D, D), lambda i: (0, 0))
_b_spec = pl.BlockSpec((1, D), lambda i: (0, 0))
_out_struct = jax.ShapeDtypeStruct((N_PAD, D), jnp.float32)

_mm1 = pl.pallas_call(
    _mm1_body, grid=_GRID,
    in_specs=[_row_spec, _deg_spec, _w_spec],
    out_specs=_row_spec, out_shape=_out_struct)

_mid = pl.pallas_call(
    _mid_body, grid=_GRID,
    in_specs=[_row_spec, _row_spec, _deg_spec, _deg_spec, _b_spec, _w_spec],
    out_specs=_row_spec, out_shape=_out_struct)

_FR = 2000
_fin = pl.pallas_call(
    _fin_body, grid=(N_NODES // _FR,),
    in_specs=[pl.BlockSpec((_FR, D), lambda i: (i, 0)),
              pl.BlockSpec((_FR, D), lambda i: (i, 0)),
              pl.BlockSpec((_FR, 1), lambda i: (i, 0)),
              pl.BlockSpec((1, D), lambda i: (0, 0))],
    out_specs=pl.BlockSpec((_FR, D), lambda i: (i, 0)),
    out_shape=jax.ShapeDtypeStruct((N_NODES, D), jnp.float32))


def kernel(features, edge_index, W1, b1, W2, b2):
    ei32 = edge_index.astype(jnp.int32)
    pad = N_NODES + jnp.arange(E_PAD - N_EDGES, dtype=jnp.int32) % (N_PAD - N_NODES)
    ei = jnp.concatenate([ei32, jnp.stack([pad, pad])], axis=1)
    ei = ei.reshape(2, E_PAD // EB, EB).transpose(1, 0, 2)
    x = jnp.pad(features.astype(jnp.float32), ((0, N_PAD - N_NODES), (0, 0)))
    degs = _deg_kernel(ei)
    dgo = degs[0].reshape(N_PAD, 1)
    dgi = degs[1].reshape(N_PAD, 1)
    h1s = _mm1(x, dgo, W1)
    agg1 = _agg_kernel(h1s, ei)
    h2s = _mid(agg1[0], agg1[1], dgo, dgi, b1.reshape(1, D), W2)
    agg2 = _agg_kernel(h2s, ei)
    return _fin(agg2[0], agg2[1], dgi, b2.reshape(1, D))


# speedup vs baseline: 1.0661x; 1.0125x over previous
import jax
import jax.numpy as jnp
from jax import lax
from jax.experimental import pallas as pl
from jax.experimental.pallas import tpu as pltpu
from jax.experimental.pallas import tpu_sc as plsc

N_NODES = 10000
N_PAD = 10112
N_EDGES = 320000
E_PAD = 327680
D = 128

NC, NS = 2, 16
NW = NC * NS
EB = 128
EROWS = E_PAD // NW // EB
DEG_ROWS = E_PAD // NS // EB
RPT = N_PAD // NS
DEG_Q = 16

_MESH = plsc.VectorSubcoreMesh(core_axis_name="c", subcore_axis_name="s")


def _deg_body(ei, degs, idx_v, ones_v, zeros_v, sd, acc):
    c = lax.axis_index("c")
    s = lax.axis_index("s")
    zv = jnp.zeros((16,), jnp.float32)
    ov = jnp.ones((16,), jnp.float32)
    for j in range(64 // 16):
        zeros_v[pl.ds(j * 16, 16)] = zv
    for j in range(EB // 16):
        ones_v[pl.ds(j * 16, 16)] = ov
    for m in range(9):
        pltpu.sync_copy(zeros_v, acc.at[pl.ds(s * RPT + m * 64, 64)])
    pltpu.sync_copy(zeros_v.at[pl.ds(0, 56)], acc.at[pl.ds(s * RPT + 576, 56)])
    plsc.subcore_barrier()
    pltpu.sync_copy(ei.at[pl.ds(s * DEG_ROWS, DEG_ROWS)], idx_v)

    def body(j, carry):
        for b in range(DEG_Q):
            k = j * DEG_Q + b
            pltpu.async_copy(ones_v, acc.at[idx_v.at[k, c]], sd, add=True)
        for b in range(DEG_Q):
            k = j * DEG_Q + b
            pltpu.make_async_copy(ones_v, acc.at[idx_v.at[k, c]], sd).wait()
        return carry

    lax.fori_loop(0, DEG_ROWS // DEG_Q, body, 0)
    plsc.subcore_barrier()
    pltpu.sync_copy(acc.at[pl.ds(s * RPT, RPT)], degs.at[c, pl.ds(s * RPT, RPT)])


_deg_kernel = pl.kernel(
    _deg_body,
    out_type=jax.ShapeDtypeStruct((NC, N_PAD), jnp.float32),
    mesh=_MESH,
    compiler_params=pltpu.CompilerParams(use_tc_tiling_on_sc=False),
    scratch_types=[
        pltpu.VMEM((DEG_ROWS, 2, EB), jnp.int32),
        pltpu.VMEM((EB,), jnp.float32),
        pltpu.VMEM((64,), jnp.float32),
        pltpu.SemaphoreType.DMA,
        pltpu.VMEM_SHARED((N_PAD,), jnp.float32),
    ],
)


def _agg_body(h, ei, out, idxr, rows,
              si0, si1, si2, sg0, sg1, sg2, ss0, ss1, ss2, acc):
    c = lax.axis_index("c")
    s = lax.axis_index("s")
    w = c * NS + s
    base = w * EROWS
    si = [si0, si1, si2]
    sg = [sg0, sg1, sg2]
    ss = [ss0, ss1, ss2]
    zv = jnp.zeros((16,), jnp.float32)

    def idx_load(k, b):
        pltpu.async_copy(ei.at[base + k], idxr.at[b], si[b])

    def idx_wait(k, b):
        pltpu.make_async_copy(ei.at[base + k], idxr.at[b], si[b]).wait()

    def g_issue(b):
        pltpu.async_copy(h.at[idxr.at[b, 0]], rows.at[b], sg[b])

    def g_wait(b):
        pltpu.make_async_copy(h.at[idxr.at[b, 0]], rows.at[b], sg[b]).wait()

    def s_issue(b):
        pltpu.async_copy(rows.at[b], acc.at[idxr.at[b, 1]], ss[b], add=True)

    def s_wait(b):
        pltpu.make_async_copy(rows.at[b], acc.at[idxr.at[b, 1]], ss[b]).wait()

    for b in range(3):
        idx_load(b, b)

    def zero_rows(i, carry):
        for j in range(D // 16):
            rows[2, i, pl.ds(j * 16, 16)] = zv
        return carry

    lax.fori_loop(0, EB, zero_rows, 0)
    for b in range(2):
        idx_wait(b, b)
        g_issue(b)
    for m in range(4):
        pltpu.sync_copy(rows.at[2], acc.at[pl.ds(s * RPT + m * EB, EB)])
    pltpu.sync_copy(rows.at[2, pl.ds(0, 120)],
                    acc.at[pl.ds(s * RPT + 512, 120)])
    plsc.subcore_barrier()

    for b in range(2):
        g_wait(b)
        s_issue(b)
        idx_load(b + 3, b)
    idx_wait(2, 2)
    g_issue(2)
    g_wait(2)
    s_issue(2)
    idx_load(5, 2)

    def body(j, carry):
        for b in range(3):
            k = 3 * j + b
            s_wait(b)
            idx_wait(k, b)
            g_issue(b)
            g_wait(b)
            s_issue(b)
            idx_load(k + 3, b)
        return carry

    lax.fori_loop(1, EROWS // 3 - 1, body, 0)
    for k in range(EROWS - 5, EROWS):
        b = k % 3
        s_wait(b)
        idx_wait(k, b)
        g_issue(b)
        g_wait(b)
        s_issue(b)
        if k + 3 < EROWS:
            idx_load(k + 3, b)
    for b in (2, 0, 1):
        s_wait(b)
    plsc.subcore_barrier()
    pltpu.sync_copy(acc.at[pl.ds(s * RPT, RPT)], out.at[c, pl.ds(s * RPT, RPT)])


_agg_kernel = pl.kernel(
    _agg_body,
    out_type=jax.ShapeDtypeStruct((NC, N_PAD, D), jnp.float32),
    mesh=_MESH,
    compiler_params=pltpu.CompilerParams(use_tc_tiling_on_sc=False),
    scratch_types=[
        pltpu.VMEM((3, 2, EB), jnp.int32),
        pltpu.VMEM((3, EB, D), jnp.float32),
        pltpu.SemaphoreType.DMA,
        pltpu.SemaphoreType.DMA,
        pltpu.SemaphoreType.DMA,
        pltpu.SemaphoreType.DMA,
        pltpu.SemaphoreType.DMA,
        pltpu.SemaphoreType.DMA,
        pltpu.SemaphoreType.DMA,
        pltpu.SemaphoreType.DMA,
        pltpu.SemaphoreType.DMA,
        pltpu.VMEM_SHARED((N_PAD, D), jnp.float32),
    ],
)


def _norm(deg):
    return jnp.where(deg > 0, lax.rsqrt(jnp.maximum(deg, 1.0)), 0.0)


def _mm1_body(x_ref, dgo_ref, w_ref, o_ref):
    nsrc = _norm(dgo_ref[...])
    o_ref[...] = jnp.dot(x_ref[...] * nsrc, w_ref[...],
                         preferred_element_type=jnp.float32)


def _mid_body(a0_ref, a1_ref, dgo_ref, dgi_ref, b1_ref, w_ref, o_ref):
    ndst = _norm(dgi_ref[...])
    nsrc = _norm(dgo_ref[...])
    t = jnp.maximum((a0_ref[...] + a1_ref[...]) * ndst + b1_ref[...], 0.0)
    o_ref[...] = jnp.dot(t * nsrc, w_ref[...],
                         preferred_element_type=jnp.float32)


def _fin_body(a0_ref, a1_ref, dgi_ref, b2_ref, o_ref):
    ndst = _norm(dgi_ref[...])
    o_ref[...] = (a0_ref[...] + a1_ref[...]) * ndst + b2_ref[...]


_R = 1264
_GRID = (N_PAD // _R,)
_row_spec = pl.BlockSpec((_R, D), lambda i: (i, 0))
_deg_spec = pl.BlockSpec((_R, 1), lambda i: (i, 0))
_w_spec = pl.BlockSpec((D, D), lambda i: (0, 0))
_b_spec = pl.BlockSpec((1, D), lambda i: (0, 0))
_out_struct = jax.ShapeDtypeStruct((N_PAD, D), jnp.float32)

_mm1 = pl.pallas_call(
    _mm1_body, grid=_GRID,
    in_specs=[_row_spec, _deg_spec, _w_spec],
    out_specs=_row_spec, out_shape=_out_struct)

_mid = pl.pallas_call(
    _mid_body, grid=_GRID,
    in_specs=[_row_spec, _row_spec, _deg_spec, _deg_spec, _b_spec, _w_spec],
    out_specs=_row_spec, out_shape=_out_struct)

_FR = 2000
_fin = pl.pallas_call(
    _fin_body, grid=(N_NODES // _FR,),
    in_specs=[pl.BlockSpec((_FR, D), lambda i: (i, 0)),
              pl.BlockSpec((_FR, D), lambda i: (i, 0)),
              pl.BlockSpec((_FR, 1), lambda i: (i, 0)),
              pl.BlockSpec((1, D), lambda i: (0, 0))],
    out_specs=pl.BlockSpec((_FR, D), lambda i: (i, 0)),
    out_shape=jax.ShapeDtypeStruct((N_NODES, D), jnp.float32))


def kernel(features, edge_index, W1, b1, W2, b2):
    ei32 = edge_index.astype(jnp.int32)
    pad = N_NODES + jnp.arange(E_PAD - N_EDGES, dtype=jnp.int32) % (N_PAD - N_NODES)
    ei = jnp.concatenate([ei32, jnp.stack([pad, pad])], axis=1)
    ei = ei.reshape(2, E_PAD // EB, EB).transpose(1, 0, 2)
    x = jnp.pad(features.astype(jnp.float32), ((0, N_PAD - N_NODES), (0, 0)))
    degs = _deg_kernel(ei)
    dgo = degs[0].reshape(N_PAD, 1)
    dgi = degs[1].reshape(N_PAD, 1)
    h1s = _mm1(x, dgo, W1)
    agg1 = _agg_kernel(h1s, ei)
    h2s = _mid(agg1[0], agg1[1], dgo, dgi, b1.reshape(1, D), W2)
    agg2 = _agg_kernel(h2s, ei)
    return _fin(agg2[0], agg2[1], dgi, b2.reshape(1, D))
